# baseline (speedup 1.0000x reference)
import jax
import jax.numpy as jnp
from jax import lax
from jax.experimental import pallas as pl
from jax.experimental.pallas import tpu as pltpu
from jax.experimental.pallas import tpu_sc as plsc

N_NODE = 10000
D = 128
E = 320000
NC = 2
NS = 16
NW = NC * NS
CH = 320
CPT = -(-E // (NW * CH))
NG = CPT
E_PAD = NW * CPT * CH
ROWS = 128 * (-(-N_NODE // 128))
RPT = ROWS // NS
DUMMY = N_NODE


def _sc_body(feat, srci, dsti, zacc, ones, sums, cnts,
             src_c, dst_c, buf, acc_s, semg, semb):
    c = lax.axis_index("c")
    s = lax.axis_index("s")
    wid = s * NC + c
    base = s * RPT
    pltpu.sync_copy(zacc, acc_s.at[pl.ds(base, RPT)])
    plsc.subcore_barrier()

    def chunk_a(j, carry):
        pltpu.sync_copy(srci.at[wid, j], src_c)
        pltpu.sync_copy(dsti.at[wid, j], dst_c)
        pltpu.async_copy(feat.at[src_c], buf, semg).wait()
        pltpu.sync_copy(buf, acc_s.at[dst_c], add=True)
        return carry

    lax.fori_loop(0, NG, chunk_a, 0)
    plsc.subcore_barrier()
    pltpu.sync_copy(acc_s.at[pl.ds(base, RPT)], sums.at[c, pl.ds(base, RPT)])
    plsc.subcore_barrier()

    pltpu.sync_copy(zacc, acc_s.at[pl.ds(base, RPT)])
    pltpu.sync_copy(ones, buf)
    plsc.subcore_barrier()

    def chunk_b(j, carry):
        pltpu.sync_copy(dsti.at[wid, j], dst_c)
        pltpu.async_copy(buf, acc_s.at[dst_c], semb, add=True).wait()
        return carry

    lax.fori_loop(0, NG, chunk_b, 0)
    plsc.subcore_barrier()
    pltpu.sync_copy(acc_s.at[pl.ds(base, RPT)], cnts.at[c, pl.ds(base, RPT)])


@jax.jit
def _sc_segment_sum(feat, src_pad, dst_pad):
    zacc = jnp.zeros((RPT, D), jnp.float32)
    ones = jnp.ones((CH, D), jnp.float32)
    mesh = plsc.VectorSubcoreMesh(core_axis_name="c", subcore_axis_name="s")
    f = pl.kernel(
        _sc_body,
        out_type=(
            jax.ShapeDtypeStruct((NC, ROWS, D), jnp.float32),
            jax.ShapeDtypeStruct((NC, ROWS, D), jnp.float32),
        ),
        mesh=mesh,
        scratch_types=[
            pltpu.VMEM((CH,), jnp.int32),
            pltpu.VMEM((CH,), jnp.int32),
            pltpu.VMEM((CH, D), jnp.float32),
            pltpu.VMEM_SHARED((ROWS, D), jnp.float32),
            pltpu.SemaphoreType.DMA,
            pltpu.SemaphoreType.DMA,
        ],
    )
    return f(feat, src_pad, dst_pad, zacc, ones)


def _tc_body(sf, cf, sc_, cc, sb, cb, wf, bf, wc, bc, wb, bb, hu, hi):
    def rel(s_ref, c_ref, w_ref, b_ref):
        s = s_ref[0] + s_ref[1]
        cnt = c_ref[0, :, 0:1] + c_ref[1, :, 0:1]
        mean = s * (1.0 / jnp.maximum(cnt, 1.0))
        h = lax.dot_general(mean, w_ref[...], (((1,), (1,)), ((), ())),
                            preferred_element_type=jnp.float32)
        return jnp.where(cnt > 0.0, h + b_ref[...], 0.0)

    hu[...] = rel(sf, cf, wf, bf) + rel(sb, cb, wb, bb)
    hi[...] = rel(sc_, cc, wc, bc)


@jax.jit
def _tc_combine(Sf, Cf, Sc, Cc, Sb, Cb, Wf, bf, Wc, bc, Wb, bb):
    BR = 1000
    grid = (N_NODE // BR,)
    s_spec = pl.BlockSpec((NC, BR, D), lambda i: (0, i, 0))
    w_spec = pl.BlockSpec((D, D), lambda i: (0, 0))
    b_spec = pl.BlockSpec((1, D), lambda i: (0, 0))
    o_spec = pl.BlockSpec((BR, D), lambda i: (i, 0))
    return pl.pallas_call(
        _tc_body,
        grid=grid,
        in_specs=[s_spec, s_spec, s_spec, s_spec, s_spec, s_spec,
                  w_spec, b_spec, w_spec, b_spec, w_spec, b_spec],
        out_specs=[o_spec, o_spec],
        out_shape=[
            jax.ShapeDtypeStruct((N_NODE, D), jnp.float32),
            jax.ShapeDtypeStruct((N_NODE, D), jnp.float32),
        ],
    )(Sf, Cf, Sc, Cc, Sb, Cb,
      Wf, bf.reshape(1, D), Wc, bc.reshape(1, D), Wb, bb.reshape(1, D))


def _pad_edges(edge_index):
    src = jnp.concatenate(
        [edge_index[0], jnp.zeros((E_PAD - E,), jnp.int32)])
    dst = jnp.concatenate(
        [edge_index[1], jnp.full((E_PAD - E,), DUMMY, jnp.int32)])
    return src.reshape(NW, CPT, CH), dst.reshape(NW, CPT, CH)


def kernel(feat_user, feat_item, W_follows, b_follows, W_clicks, b_clicks,
           W_bought, b_bought, edge_index_follows, edge_index_clicks,
           edge_index_bought):
    sf, df = _pad_edges(edge_index_follows)
    sc_, dc = _pad_edges(edge_index_clicks)
    sb, db = _pad_edges(edge_index_bought)
    Sf, Cf = _sc_segment_sum(feat_user, sf, df)
    Sc, Cc = _sc_segment_sum(feat_user, sc_, dc)
    Sb, Cb = _sc_segment_sum(feat_item, sb, db)
    h_user, h_item = _tc_combine(Sf, Cf, Sc, Cc, Sb, Cb,
                                 W_follows, b_follows, W_clicks, b_clicks,
                                 W_bought, b_bought)
    return (h_user, h_item)


# speedup vs baseline: 1.2184x; 1.2184x over previous
import jax
import jax.numpy as jnp
from jax import lax
from jax.experimental import pallas as pl
from jax.experimental.pallas import tpu as pltpu
from jax.experimental.pallas import tpu_sc as plsc

N_NODE = 10000
D = 128
E = 320000
NC = 2
NS = 16
NW = NC * NS
CH = 128
G = 8
TOTC = -(-E // CH)
TOTC = NS * G * (-(-TOTC // (NS * G)))
E_PAD = TOTC * CH
CPT_F = 104
CPT_S = TOTC // NS - CPT_F
NG_F = CPT_F // G
NG_S = CPT_S // G
ROWS = 128 * (-(-N_NODE // 128))
RPT = ROWS // NS
DUMMY = N_NODE


def _sc_body(feat, srci, dsti, zacc, ones, sums, cnts,
             src_g, dst_g, buf0, buf1, acc_s, sem0, sem1, semb):
    c = lax.axis_index("c")
    s = lax.axis_index("s")
    base = s * RPT
    ng = jnp.where(c == 1, NG_F, NG_S)
    chunk0 = jnp.where(c == 1, s * CPT_F, NS * CPT_F + s * CPT_S)
    _ = jnp.int32(0)
    pltpu.sync_copy(zacc, acc_s.at[pl.ds(base, RPT)])
    plsc.subcore_barrier()

    bufs = (buf0, buf1)
    sems = (sem0, sem1)

    def grp_a(g, carry):
        row = chunk0 + g * G
        pltpu.sync_copy(srci.at[pl.ds(row, G)], src_g)
        pltpu.sync_copy(dsti.at[pl.ds(row, G)], dst_g)
        pend = pltpu.async_copy(feat.at[src_g.at[0]], buf0, sem0)
        for j in range(G):
            if j + 1 < G:
                nxt = pltpu.async_copy(feat.at[src_g.at[j + 1]],
                                       bufs[(j + 1) % 2], sems[(j + 1) % 2])
            pend.wait()
            pltpu.sync_copy(bufs[j % 2], acc_s.at[dst_g.at[j]], add=True)
            if j + 1 < G:
                pend = nxt
        return carry

    lax.fori_loop(0, ng, grp_a, 0)
    plsc.subcore_barrier()
    pltpu.sync_copy(acc_s.at[pl.ds(base, RPT)], sums.at[c, pl.ds(base, RPT)])
    plsc.subcore_barrier()

    pltpu.sync_copy(zacc, acc_s.at[pl.ds(base, RPT)])
    pltpu.sync_copy(ones, buf0)
    plsc.subcore_barrier()

    def grp_b(g, carry):
        row = chunk0 + g * G
        pltpu.sync_copy(dsti.at[pl.ds(row, G)], dst_g)
        descs = [pltpu.async_copy(buf0, acc_s.at[dst_g.at[j]], semb, add=True)
                 for j in range(G)]
        for d in descs:
            d.wait()
        return carry

    lax.fori_loop(0, ng, grp_b, 0)
    plsc.subcore_barrier()
    pltpu.sync_copy(acc_s.at[pl.ds(base, RPT)], cnts.at[c, pl.ds(base, RPT)])


@jax.jit
def _sc_segment_sum(feat, src_pad, dst_pad):
    zacc = jnp.zeros((RPT, D), jnp.float32)
    ones = jnp.ones((CH, D), jnp.float32)
    mesh = plsc.VectorSubcoreMesh(core_axis_name="c", subcore_axis_name="s")
    f = pl.kernel(
        _sc_body,
        out_type=(
            jax.ShapeDtypeStruct((NC, ROWS, D), jnp.float32),
            jax.ShapeDtypeStruct((NC, ROWS, D), jnp.float32),
        ),
        mesh=mesh,
        scratch_types=[
            pltpu.VMEM((G, CH), jnp.int32),
            pltpu.VMEM((G, CH), jnp.int32),
            pltpu.VMEM((CH, D), jnp.float32),
            pltpu.VMEM((CH, D), jnp.float32),
            pltpu.VMEM_SHARED((ROWS, D), jnp.float32),
            pltpu.SemaphoreType.DMA,
            pltpu.SemaphoreType.DMA,
            pltpu.SemaphoreType.DMA,
        ],
    )
    return f(feat, src_pad, dst_pad, zacc, ones)


def _tc_body(sf, cf, sc_, cc, sb, cb, wf, bf, wc, bc, wb, bb, hu, hi):
    def rel(s_ref, c_ref, w_ref, b_ref):
        s = s_ref[0] + s_ref[1]
        cnt = c_ref[0, :, 0:1] + c_ref[1, :, 0:1]
        mean = s * (1.0 / jnp.maximum(cnt, 1.0))
        h = lax.dot_general(mean, w_ref[...], (((1,), (1,)), ((), ())),
                            preferred_element_type=jnp.float32)
        return jnp.where(cnt > 0.0, h + b_ref[...], 0.0)

    hu[...] = rel(sf, cf, wf, bf) + rel(sb, cb, wb, bb)
    hi[...] = rel(sc_, cc, wc, bc)


@jax.jit
def _tc_combine(Sf, Cf, Sc, Cc, Sb, Cb, Wf, bf, Wc, bc, Wb, bb):
    BR = 1000
    grid = (N_NODE // BR,)
    s_spec = pl.BlockSpec((NC, BR, D), lambda i: (0, i, 0))
    w_spec = pl.BlockSpec((D, D), lambda i: (0, 0))
    b_spec = pl.BlockSpec((1, D), lambda i: (0, 0))
    o_spec = pl.BlockSpec((BR, D), lambda i: (i, 0))
    return pl.pallas_call(
        _tc_body,
        grid=grid,
        in_specs=[s_spec, s_spec, s_spec, s_spec, s_spec, s_spec,
                  w_spec, b_spec, w_spec, b_spec, w_spec, b_spec],
        out_specs=[o_spec, o_spec],
        out_shape=[
            jax.ShapeDtypeStruct((N_NODE, D), jnp.float32),
            jax.ShapeDtypeStruct((N_NODE, D), jnp.float32),
        ],
    )(Sf, Cf, Sc, Cc, Sb, Cb,
      Wf, bf.reshape(1, D), Wc, bc.reshape(1, D), Wb, bb.reshape(1, D))


def _pad_edges(edge_index):
    src = jnp.concatenate(
        [edge_index[0], jnp.zeros((E_PAD - E,), jnp.int32)])
    dst = jnp.concatenate(
        [edge_index[1], jnp.full((E_PAD - E,), DUMMY, jnp.int32)])
    return src.reshape(TOTC, CH), dst.reshape(TOTC, CH)


def kernel(feat_user, feat_item, W_follows, b_follows, W_clicks, b_clicks,
           W_bought, b_bought, edge_index_follows, edge_index_clicks,
           edge_index_bought):
    sf, df = _pad_edges(edge_index_follows)
    sc_, dc = _pad_edges(edge_index_clicks)
    sb, db = _pad_edges(edge_index_bought)
    Sf, Cf = _sc_segment_sum(feat_user, sf, df)
    Sc, Cc = _sc_segment_sum(feat_user, sc_, dc)
    Sb, Cb = _sc_segment_sum(feat_item, sb, db)
    h_user, h_item = _tc_combine(Sf, Cf, Sc, Cc, Sb, Cb,
                                 W_follows, b_follows, W_clicks, b_clicks,
                                 W_bought, b_bought)
    return (h_user, h_item)


# speedup vs baseline: 1.2557x; 1.0306x over previous
import jax
import jax.numpy as jnp
from jax import lax
from jax.experimental import pallas as pl
from jax.experimental.pallas import tpu as pltpu
from jax.experimental.pallas import tpu_sc as plsc

N_NODE = 10000
D = 128
E = 320000
NC = 2
NS = 16
NW = NC * NS
CH = 128
G = 8
TOTC = -(-E // CH)
TOTC = NS * G * (-(-TOTC // (NS * G)))
E_PAD = TOTC * CH
CPT_F = 112
CPT_S = TOTC // NS - CPT_F
NG_F = CPT_F // G
NG_S = CPT_S // G
ROWS = 128 * (-(-N_NODE // 128))
RPT = ROWS // NS
DUMMY = N_NODE


def _sc_body(feat, srci, dsti, zacc, ones, sums, cnts,
             src_g, dst_g, buf0, buf1, acc_s, sem0, sem1, semb):
    c = lax.axis_index("c")
    s = lax.axis_index("s")
    base = s * RPT
    ng = jnp.where(c == 1, NG_F, NG_S)
    chunk0 = jnp.where(c == 1, s * CPT_F, NS * CPT_F + s * CPT_S)
    _ = jnp.int32(0)
    pltpu.sync_copy(zacc, acc_s.at[pl.ds(base, RPT)])
    plsc.subcore_barrier()

    bufs = (buf0, buf1)
    sems = (sem0, sem1)

    def grp_a(g, carry):
        row = chunk0 + g * G
        pltpu.sync_copy(srci.at[pl.ds(row, G)], src_g)
        pltpu.sync_copy(dsti.at[pl.ds(row, G)], dst_g)
        pend = pltpu.async_copy(feat.at[src_g.at[0]], buf0, sem0)
        for j in range(G):
            if j + 1 < G:
                nxt = pltpu.async_copy(feat.at[src_g.at[j + 1]],
                                       bufs[(j + 1) % 2], sems[(j + 1) % 2])
            pend.wait()
            pltpu.sync_copy(bufs[j % 2], acc_s.at[dst_g.at[j]], add=True)
            if j + 1 < G:
                pend = nxt
        return carry

    lax.fori_loop(0, ng, grp_a, 0)
    plsc.subcore_barrier()
    pltpu.sync_copy(acc_s.at[pl.ds(base, RPT)], sums.at[c, pl.ds(base, RPT)])
    plsc.subcore_barrier()

    pltpu.sync_copy(zacc, acc_s.at[pl.ds(base, RPT)])
    pltpu.sync_copy(ones, buf0)
    plsc.subcore_barrier()

    def grp_b(g, carry):
        row = chunk0 + g * G
        pltpu.sync_copy(dsti.at[pl.ds(row, G)], dst_g)
        descs = [pltpu.async_copy(buf0, acc_s.at[dst_g.at[j]], semb, add=True)
                 for j in range(G)]
        for d in descs:
            d.wait()
        return carry

    lax.fori_loop(0, ng, grp_b, 0)
    plsc.subcore_barrier()
    pltpu.sync_copy(acc_s.at[pl.ds(base, RPT)], cnts.at[c, pl.ds(base, RPT)])


@jax.jit
def _sc_segment_sum(feat, src_pad, dst_pad):
    zacc = jnp.zeros((RPT, D), jnp.float32)
    ones = jnp.ones((CH, D), jnp.float32)
    mesh = plsc.VectorSubcoreMesh(core_axis_name="c", subcore_axis_name="s")
    f = pl.kernel(
        _sc_body,
        out_type=(
            jax.ShapeDtypeStruct((NC, ROWS, D), jnp.float32),
            jax.ShapeDtypeStruct((NC, ROWS, D), jnp.float32),
        ),
        mesh=mesh,
        scratch_types=[
            pltpu.VMEM((G, CH), jnp.int32),
            pltpu.VMEM((G, CH), jnp.int32),
            pltpu.VMEM((CH, D), jnp.float32),
            pltpu.VMEM((CH, D), jnp.float32),
            pltpu.VMEM_SHARED((ROWS, D), jnp.float32),
            pltpu.SemaphoreType.DMA,
            pltpu.SemaphoreType.DMA,
            pltpu.SemaphoreType.DMA,
        ],
    )
    return f(feat, src_pad, dst_pad, zacc, ones)


def _tc_body(sf, cf, sc_, cc, sb, cb, wf, bf, wc, bc, wb, bb, hu, hi):
    def rel(s_ref, c_ref, w_ref, b_ref):
        s = s_ref[0] + s_ref[1]
        cnt = c_ref[0, :, 0:1] + c_ref[1, :, 0:1]
        mean = s * (1.0 / jnp.maximum(cnt, 1.0))
        h = lax.dot_general(mean, w_ref[...], (((1,), (1,)), ((), ())),
                            preferred_element_type=jnp.float32)
        return jnp.where(cnt > 0.0, h + b_ref[...], 0.0)

    hu[...] = rel(sf, cf, wf, bf) + rel(sb, cb, wb, bb)
    hi[...] = rel(sc_, cc, wc, bc)


@jax.jit
def _tc_combine(Sf, Cf, Sc, Cc, Sb, Cb, Wf, bf, Wc, bc, Wb, bb):
    BR = 1000
    grid = (N_NODE // BR,)
    s_spec = pl.BlockSpec((NC, BR, D), lambda i: (0, i, 0))
    w_spec = pl.BlockSpec((D, D), lambda i: (0, 0))
    b_spec = pl.BlockSpec((1, D), lambda i: (0, 0))
    o_spec = pl.BlockSpec((BR, D), lambda i: (i, 0))
    return pl.pallas_call(
        _tc_body,
        grid=grid,
        in_specs=[s_spec, s_spec, s_spec, s_spec, s_spec, s_spec,
                  w_spec, b_spec, w_spec, b_spec, w_spec, b_spec],
        out_specs=[o_spec, o_spec],
        out_shape=[
            jax.ShapeDtypeStruct((N_NODE, D), jnp.float32),
            jax.ShapeDtypeStruct((N_NODE, D), jnp.float32),
        ],
    )(Sf, Cf, Sc, Cc, Sb, Cb,
      Wf, bf.reshape(1, D), Wc, bc.reshape(1, D), Wb, bb.reshape(1, D))


def _pad_edges(edge_index):
    src = jnp.concatenate(
        [edge_index[0], jnp.zeros((E_PAD - E,), jnp.int32)])
    dst = jnp.concatenate(
        [edge_index[1], jnp.full((E_PAD - E,), DUMMY, jnp.int32)])
    return src.reshape(TOTC, CH), dst.reshape(TOTC, CH)


def kernel(feat_user, feat_item, W_follows, b_follows, W_clicks, b_clicks,
           W_bought, b_bought, edge_index_follows, edge_index_clicks,
           edge_index_bought):
    sf, df = _pad_edges(edge_index_follows)
    sc_, dc = _pad_edges(edge_index_clicks)
    sb, db = _pad_edges(edge_index_bought)
    Sf, Cf = _sc_segment_sum(feat_user, sf, df)
    Sc, Cc = _sc_segment_sum(feat_user, sc_, dc)
    Sb, Cb = _sc_segment_sum(feat_item, sb, db)
    h_user, h_item = _tc_combine(Sf, Cf, Sc, Cc, Sb, Cb,
                                 W_follows, b_follows, W_clicks, b_clicks,
                                 W_bought, b_bought)
    return (h_user, h_item)


# speedup vs baseline: 1.2937x; 1.0302x over previous
import jax
import jax.numpy as jnp
from jax import lax
from jax.experimental import pallas as pl
from jax.experimental.pallas import tpu as pltpu
from jax.experimental.pallas import tpu_sc as plsc

N_NODE = 10000
D = 128
E = 320000
NC = 2
NS = 16
NW = NC * NS
CH = 128
G = 8
TOTC = -(-E // CH)
TOTC = NS * G * (-(-TOTC // (NS * G)))
E_PAD = TOTC * CH
CPT_F = 120
CPT_S = TOTC // NS - CPT_F
NG_F = CPT_F // G
NG_S = CPT_S // G
ROWS = 128 * (-(-N_NODE // 128))
RPT = ROWS // NS
DUMMY = N_NODE


def _sc_body(feat, srci, dsti, zacc, ones, sums, cnts,
             src_g, dst_g, buf0, buf1, acc_s, sem0, sem1, semb):
    c = lax.axis_index("c")
    s = lax.axis_index("s")
    base = s * RPT
    ng = jnp.where(c == 1, NG_F, NG_S)
    chunk0 = jnp.where(c == 1, s * CPT_F, NS * CPT_F + s * CPT_S)
    _ = jnp.int32(0)
    pltpu.sync_copy(zacc, acc_s.at[pl.ds(base, RPT)])
    plsc.subcore_barrier()

    bufs = (buf0, buf1)
    sems = (sem0, sem1)

    def grp_a(g, carry):
        row = chunk0 + g * G
        pltpu.sync_copy(srci.at[pl.ds(row, G)], src_g)
        pltpu.sync_copy(dsti.at[pl.ds(row, G)], dst_g)
        pend = pltpu.async_copy(feat.at[src_g.at[0]], buf0, sem0)
        for j in range(G):
            if j + 1 < G:
                nxt = pltpu.async_copy(feat.at[src_g.at[j + 1]],
                                       bufs[(j + 1) % 2], sems[(j + 1) % 2])
            pend.wait()
            pltpu.sync_copy(bufs[j % 2], acc_s.at[dst_g.at[j]], add=True)
            if j + 1 < G:
                pend = nxt
        return carry

    lax.fori_loop(0, ng, grp_a, 0)
    plsc.subcore_barrier()
    pltpu.sync_copy(acc_s.at[pl.ds(base, RPT)], sums.at[c, pl.ds(base, RPT)])
    plsc.subcore_barrier()

    pltpu.sync_copy(zacc, acc_s.at[pl.ds(base, RPT)])
    pltpu.sync_copy(ones, buf0)
    plsc.subcore_barrier()

    def grp_b(g, carry):
        row = chunk0 + g * G
        pltpu.sync_copy(dsti.at[pl.ds(row, G)], dst_g)
        descs = [pltpu.async_copy(buf0, acc_s.at[dst_g.at[j]], semb, add=True)
                 for j in range(G)]
        for d in descs:
            d.wait()
        return carry

    lax.fori_loop(0, ng, grp_b, 0)
    plsc.subcore_barrier()
    pltpu.sync_copy(acc_s.at[pl.ds(base, RPT)], cnts.at[c, pl.ds(base, RPT)])


@jax.jit
def _sc_segment_sum(feat, src_pad, dst_pad):
    zacc = jnp.zeros((RPT, D), jnp.float32)
    ones = jnp.ones((CH, D), jnp.float32)
    mesh = plsc.VectorSubcoreMesh(core_axis_name="c", subcore_axis_name="s")
    f = pl.kernel(
        _sc_body,
        out_type=(
            jax.ShapeDtypeStruct((NC, ROWS, D), jnp.float32),
            jax.ShapeDtypeStruct((NC, ROWS, D), jnp.float32),
        ),
        mesh=mesh,
        scratch_types=[
            pltpu.VMEM((G, CH), jnp.int32),
            pltpu.VMEM((G, CH), jnp.int32),
            pltpu.VMEM((CH, D), jnp.float32),
            pltpu.VMEM((CH, D), jnp.float32),
            pltpu.VMEM_SHARED((ROWS, D), jnp.float32),
            pltpu.SemaphoreType.DMA,
            pltpu.SemaphoreType.DMA,
            pltpu.SemaphoreType.DMA,
        ],
    )
    return f(feat, src_pad, dst_pad, zacc, ones)


def _tc_body(sf, cf, sc_, cc, sb, cb, wf, bf, wc, bc, wb, bb, hu, hi):
    def rel(s_ref, c_ref, w_ref, b_ref):
        s = s_ref[0] + s_ref[1]
        cnt = c_ref[0, :, 0:1] + c_ref[1, :, 0:1]
        mean = s * (1.0 / jnp.maximum(cnt, 1.0))
        h = lax.dot_general(mean, w_ref[...], (((1,), (1,)), ((), ())),
                            preferred_element_type=jnp.float32)
        return jnp.where(cnt > 0.0, h + b_ref[...], 0.0)

    hu[...] = rel(sf, cf, wf, bf) + rel(sb, cb, wb, bb)
    hi[...] = rel(sc_, cc, wc, bc)


@jax.jit
def _tc_combine(Sf, Cf, Sc, Cc, Sb, Cb, Wf, bf, Wc, bc, Wb, bb):
    BR = 1000
    grid = (N_NODE // BR,)
    s_spec = pl.BlockSpec((NC, BR, D), lambda i: (0, i, 0))
    w_spec = pl.BlockSpec((D, D), lambda i: (0, 0))
    b_spec = pl.BlockSpec((1, D), lambda i: (0, 0))
    o_spec = pl.BlockSpec((BR, D), lambda i: (i, 0))
    return pl.pallas_call(
        _tc_body,
        grid=grid,
        in_specs=[s_spec, s_spec, s_spec, s_spec, s_spec, s_spec,
                  w_spec, b_spec, w_spec, b_spec, w_spec, b_spec],
        out_specs=[o_spec, o_spec],
        out_shape=[
            jax.ShapeDtypeStruct((N_NODE, D), jnp.float32),
            jax.ShapeDtypeStruct((N_NODE, D), jnp.float32),
        ],
    )(Sf, Cf, Sc, Cc, Sb, Cb,
      Wf, bf.reshape(1, D), Wc, bc.reshape(1, D), Wb, bb.reshape(1, D))


def _pad_edges(edge_index):
    src = jnp.concatenate(
        [edge_index[0], jnp.zeros((E_PAD - E,), jnp.int32)])
    dst = jnp.concatenate(
        [edge_index[1], jnp.full((E_PAD - E,), DUMMY, jnp.int32)])
    return src.reshape(TOTC, CH), dst.reshape(TOTC, CH)


def kernel(feat_user, feat_item, W_follows, b_follows, W_clicks, b_clicks,
           W_bought, b_bought, edge_index_follows, edge_index_clicks,
           edge_index_bought):
    sf, df = _pad_edges(edge_index_follows)
    sc_, dc = _pad_edges(edge_index_clicks)
    sb, db = _pad_edges(edge_index_bought)
    Sf, Cf = _sc_segment_sum(feat_user, sf, df)
    Sc, Cc = _sc_segment_sum(feat_user, sc_, dc)
    Sb, Cb = _sc_segment_sum(feat_item, sb, db)
    h_user, h_item = _tc_combine(Sf, Cf, Sc, Cc, Sb, Cb,
                                 W_follows, b_follows, W_clicks, b_clicks,
                                 W_bought, b_bought)
    return (h_user, h_item)


# speedup vs baseline: 1.3345x; 1.0316x over previous
import jax
import jax.numpy as jnp
from jax import lax
from jax.experimental import pallas as pl
from jax.experimental.pallas import tpu as pltpu
from jax.experimental.pallas import tpu_sc as plsc

N_NODE = 10000
D = 128
E = 320000
NC = 2
NS = 16
NW = NC * NS
CH = 128
G = 8
TOTC = -(-E // CH)
TOTC = NS * G * (-(-TOTC // (NS * G)))
E_PAD = TOTC * CH
CPT_F = 128
CPT_S = TOTC // NS - CPT_F
NG_F = CPT_F // G
NG_S = CPT_S // G
ROWS = 128 * (-(-N_NODE // 128))
RPT = ROWS // NS
DUMMY = N_NODE


def _sc_body(feat, srci, dsti, zacc, ones, sums, cnts,
             src_g, dst_g, buf0, buf1, acc_s, sem0, sem1, semb):
    c = lax.axis_index("c")
    s = lax.axis_index("s")
    base = s * RPT
    ng = jnp.where(c == 1, NG_F, NG_S)
    chunk0 = jnp.where(c == 1, s * CPT_F, NS * CPT_F + s * CPT_S)
    _ = jnp.int32(0)
    pltpu.sync_copy(zacc, acc_s.at[pl.ds(base, RPT)])
    plsc.subcore_barrier()

    bufs = (buf0, buf1)
    sems = (sem0, sem1)

    def grp_a(g, carry):
        row = chunk0 + g * G
        pltpu.sync_copy(srci.at[pl.ds(row, G)], src_g)
        pltpu.sync_copy(dsti.at[pl.ds(row, G)], dst_g)
        pend = pltpu.async_copy(feat.at[src_g.at[0]], buf0, sem0)
        for j in range(G):
            if j + 1 < G:
                nxt = pltpu.async_copy(feat.at[src_g.at[j + 1]],
                                       bufs[(j + 1) % 2], sems[(j + 1) % 2])
            pend.wait()
            pltpu.sync_copy(bufs[j % 2], acc_s.at[dst_g.at[j]], add=True)
            if j + 1 < G:
                pend = nxt
        return carry

    lax.fori_loop(0, ng, grp_a, 0)
    plsc.subcore_barrier()
    pltpu.sync_copy(acc_s.at[pl.ds(base, RPT)], sums.at[c, pl.ds(base, RPT)])
    plsc.subcore_barrier()

    pltpu.sync_copy(zacc, acc_s.at[pl.ds(base, RPT)])
    pltpu.sync_copy(ones, buf0)
    plsc.subcore_barrier()

    def grp_b(g, carry):
        row = chunk0 + g * G
        pltpu.sync_copy(dsti.at[pl.ds(row, G)], dst_g)
        descs = [pltpu.async_copy(buf0, acc_s.at[dst_g.at[j]], semb, add=True)
                 for j in range(G)]
        for d in descs:
            d.wait()
        return carry

    lax.fori_loop(0, ng, grp_b, 0)
    plsc.subcore_barrier()
    pltpu.sync_copy(acc_s.at[pl.ds(base, RPT)], cnts.at[c, pl.ds(base, RPT)])


@jax.jit
def _sc_segment_sum(feat, src_pad, dst_pad):
    zacc = jnp.zeros((RPT, D), jnp.float32)
    ones = jnp.ones((CH, D), jnp.float32)
    mesh = plsc.VectorSubcoreMesh(core_axis_name="c", subcore_axis_name="s")
    f = pl.kernel(
        _sc_body,
        out_type=(
            jax.ShapeDtypeStruct((NC, ROWS, D), jnp.float32),
            jax.ShapeDtypeStruct((NC, ROWS, D), jnp.float32),
        ),
        mesh=mesh,
        scratch_types=[
            pltpu.VMEM((G, CH), jnp.int32),
            pltpu.VMEM((G, CH), jnp.int32),
            pltpu.VMEM((CH, D), jnp.float32),
            pltpu.VMEM((CH, D), jnp.float32),
            pltpu.VMEM_SHARED((ROWS, D), jnp.float32),
            pltpu.SemaphoreType.DMA,
            pltpu.SemaphoreType.DMA,
            pltpu.SemaphoreType.DMA,
        ],
    )
    return f(feat, src_pad, dst_pad, zacc, ones)


def _tc_body(sf, cf, sc_, cc, sb, cb, wf, bf, wc, bc, wb, bb, hu, hi):
    def rel(s_ref, c_ref, w_ref, b_ref):
        s = s_ref[0] + s_ref[1]
        cnt = c_ref[0, :, 0:1] + c_ref[1, :, 0:1]
        mean = s * (1.0 / jnp.maximum(cnt, 1.0))
        h = lax.dot_general(mean, w_ref[...], (((1,), (1,)), ((), ())),
                            preferred_element_type=jnp.float32)
        return jnp.where(cnt > 0.0, h + b_ref[...], 0.0)

    hu[...] = rel(sf, cf, wf, bf) + rel(sb, cb, wb, bb)
    hi[...] = rel(sc_, cc, wc, bc)


@jax.jit
def _tc_combine(Sf, Cf, Sc, Cc, Sb, Cb, Wf, bf, Wc, bc, Wb, bb):
    BR = 1000
    grid = (N_NODE // BR,)
    s_spec = pl.BlockSpec((NC, BR, D), lambda i: (0, i, 0))
    w_spec = pl.BlockSpec((D, D), lambda i: (0, 0))
    b_spec = pl.BlockSpec((1, D), lambda i: (0, 0))
    o_spec = pl.BlockSpec((BR, D), lambda i: (i, 0))
    return pl.pallas_call(
        _tc_body,
        grid=grid,
        in_specs=[s_spec, s_spec, s_spec, s_spec, s_spec, s_spec,
                  w_spec, b_spec, w_spec, b_spec, w_spec, b_spec],
        out_specs=[o_spec, o_spec],
        out_shape=[
            jax.ShapeDtypeStruct((N_NODE, D), jnp.float32),
            jax.ShapeDtypeStruct((N_NODE, D), jnp.float32),
        ],
    )(Sf, Cf, Sc, Cc, Sb, Cb,
      Wf, bf.reshape(1, D), Wc, bc.reshape(1, D), Wb, bb.reshape(1, D))


def _pad_edges(edge_index):
    src = jnp.concatenate(
        [edge_index[0], jnp.zeros((E_PAD - E,), jnp.int32)])
    dst = jnp.concatenate(
        [edge_index[1], jnp.full((E_PAD - E,), DUMMY, jnp.int32)])
    return src.reshape(TOTC, CH), dst.reshape(TOTC, CH)


def kernel(feat_user, feat_item, W_follows, b_follows, W_clicks, b_clicks,
           W_bought, b_bought, edge_index_follows, edge_index_clicks,
           edge_index_bought):
    sf, df = _pad_edges(edge_index_follows)
    sc_, dc = _pad_edges(edge_index_clicks)
    sb, db = _pad_edges(edge_index_bought)
    Sf, Cf = _sc_segment_sum(feat_user, sf, df)
    Sc, Cc = _sc_segment_sum(feat_user, sc_, dc)
    Sb, Cb = _sc_segment_sum(feat_item, sb, db)
    h_user, h_item = _tc_combine(Sf, Cf, Sc, Cc, Sb, Cb,
                                 W_follows, b_follows, W_clicks, b_clicks,
                                 W_bought, b_bought)
    return (h_user, h_item)
